# speedup vs baseline: 1.2742x; 1.2742x over previous
import functools

import jax
import jax.numpy as jnp
from jax import lax
from jax.experimental import pallas as pl
from jax.experimental.pallas import tpu as pltpu
from jax.experimental.pallas import tpu_sc as plsc

D_MODEL = 512
NC = 2
NS = 16
NW = NC * NS
CHUNK = 128


def _make_emb_kernel(b_total: int):
    b_per_w = b_total // NW
    n_chunks = b_per_w // CHUNK
    mesh = plsc.VectorSubcoreMesh(core_axis_name="c", subcore_axis_name="s")

    @functools.partial(
        pl.kernel,
        mesh=mesh,
        out_type=jax.ShapeDtypeStruct((b_total, D_MODEL), jnp.float32),
        scratch_types=[
            pltpu.VMEM((n_chunks, CHUNK), jnp.int32),
            pltpu.VMEM((CHUNK, D_MODEL), jnp.float32),
            pltpu.SemaphoreType.DMA,
        ],
    )
    def emb(idx_hbm, table_hbm, out_hbm, idx_v, rows_v, sem):
        wid = lax.axis_index("s") * NC + lax.axis_index("c")
        pltpu.sync_copy(idx_hbm.at[wid], idx_v)
        base = wid * b_per_w

        def chunk_body(j, carry):
            pltpu.async_copy(table_hbm.at[idx_v.at[j]], rows_v, sem).wait()
            pltpu.sync_copy(rows_v, out_hbm.at[pl.ds(base + j * CHUNK, CHUNK)])
            return carry

        lax.fori_loop(0, n_chunks, chunk_body, 0)

    return emb


def kernel(x, embedding_table):
    b, s = x.shape
    b_total = b * s
    idx = x.reshape(-1).astype(jnp.int32).reshape(NW, b_total // (NW * CHUNK), CHUNK)
    out = _make_emb_kernel(b_total)(idx, embedding_table)
    return out.reshape(b, s, D_MODEL)


# speedup vs baseline: 1.3000x; 1.0203x over previous
import functools

import jax
import jax.numpy as jnp
from jax import lax
from jax.experimental import pallas as pl
from jax.experimental.pallas import tpu as pltpu
from jax.experimental.pallas import tpu_sc as plsc

D_MODEL = 512
NC = 2
NS = 16
NW = NC * NS
CHUNK = 64


def _make_emb_kernel(b_total: int):
    b_per_w = b_total // NW
    n_chunks = b_per_w // CHUNK
    n_pairs = n_chunks // 2
    mesh = plsc.VectorSubcoreMesh(core_axis_name="c", subcore_axis_name="s")

    @functools.partial(
        pl.kernel,
        mesh=mesh,
        out_type=jax.ShapeDtypeStruct((b_total, D_MODEL), jnp.float32),
        scratch_types=[
            pltpu.VMEM((n_chunks, CHUNK), jnp.int32),
            pltpu.VMEM((CHUNK, D_MODEL), jnp.float32),
            pltpu.VMEM((CHUNK, D_MODEL), jnp.float32),
            pltpu.SemaphoreType.DMA,
            pltpu.SemaphoreType.DMA,
            pltpu.SemaphoreType.DMA,
            pltpu.SemaphoreType.DMA,
        ],
    )
    def emb(idx_hbm, table_hbm, out_hbm, idx_v, buf0, buf1, g0, g1, s0, s1):
        wid = lax.axis_index("s") * NC + lax.axis_index("c")
        pltpu.sync_copy(idx_hbm.at[wid], idx_v)
        base = wid * b_per_w

        def gather(j, buf, sem):
            pltpu.async_copy(table_hbm.at[idx_v.at[j]], buf, sem)

        def wait_gather(j, buf, sem):
            pltpu.make_async_copy(table_hbm.at[idx_v.at[j]], buf, sem).wait()

        def scatter(j, buf, sem):
            pltpu.async_copy(buf, out_hbm.at[pl.ds(base + j * CHUNK, CHUNK)], sem)

        def wait_scatter(j, buf, sem):
            pltpu.make_async_copy(
                buf, out_hbm.at[pl.ds(base + j * CHUNK, CHUNK)], sem
            ).wait()

        gather(0, buf0, g0)

        def pair_body(t, carry):
            j0 = 2 * t
            j1 = j0 + 1

            @pl.when(t > 0)
            def _():
                wait_scatter(j0 - 1, buf1, s1)

            gather(j1, buf1, g1)
            wait_gather(j0, buf0, g0)
            scatter(j0, buf0, s0)

            @pl.when(t < n_pairs - 1)
            def _():
                wait_scatter(j0, buf0, s0)
                gather(j0 + 2, buf0, g0)

            wait_gather(j1, buf1, g1)
            scatter(j1, buf1, s1)
            return carry

        lax.fori_loop(0, n_pairs, pair_body, 0)
        wait_scatter(n_chunks - 2, buf0, s0)
        wait_scatter(n_chunks - 1, buf1, s1)

    return emb


def kernel(x, embedding_table):
    b, s = x.shape
    b_total = b * s
    idx = x.reshape(-1).astype(jnp.int32).reshape(NW, b_total // (NW * CHUNK), CHUNK)
    out = _make_emb_kernel(b_total)(idx, embedding_table)
    return out.reshape(b, s, D_MODEL)


# speedup vs baseline: 4.0464x; 3.1126x over previous
import functools

import jax
import jax.numpy as jnp
from jax import lax
from jax.experimental import pallas as pl
from jax.experimental.pallas import tpu as pltpu
from jax.experimental.pallas import tpu_sc as plsc

D_MODEL = 512
NC = 2
NS = 16
NW = NC * NS
CHUNK = 64


def _make_emb_kernel(b_total: int):
    b_per_w = b_total // NW
    n_chunks = b_per_w // CHUNK
    n_pairs = n_chunks // 2
    mesh = plsc.VectorSubcoreMesh(core_axis_name="c", subcore_axis_name="s")

    @functools.partial(
        pl.kernel,
        mesh=mesh,
        out_type=jax.ShapeDtypeStruct((b_total, D_MODEL), jnp.float32),
        scratch_types=[
            pltpu.VMEM((n_chunks, CHUNK), jnp.int32),
            pltpu.VMEM((CHUNK, D_MODEL), jnp.float32),
            pltpu.VMEM((CHUNK, D_MODEL), jnp.float32),
            pltpu.SemaphoreType.DMA,
            pltpu.SemaphoreType.DMA,
            pltpu.SemaphoreType.DMA,
            pltpu.SemaphoreType.DMA,
        ],
    )
    def emb(idx_hbm, table_hbm, out_hbm, idx_v, buf0, buf1, g0, g1, s0, s1):
        wid = lax.axis_index("s") * NC + lax.axis_index("c")
        pltpu.sync_copy(idx_hbm.at[wid], idx_v)
        base = wid * b_per_w

        def gather(j, buf, sem):
            pltpu.async_copy(table_hbm.at[idx_v.at[j]], buf, sem)

        def wait_gather(j, buf, sem):
            pltpu.make_async_copy(table_hbm.at[idx_v.at[j]], buf, sem).wait()

        def scatter(j, buf, sem):
            pltpu.async_copy(buf, out_hbm.at[pl.ds(base + j * CHUNK, CHUNK)], sem)

        def wait_scatter(j, buf, sem):
            pltpu.make_async_copy(
                buf, out_hbm.at[pl.ds(base + j * CHUNK, CHUNK)], sem
            ).wait()

        gather(0, buf0, g0)

        def pair_body(t, carry):
            j0 = 2 * t
            j1 = j0 + 1

            @pl.when(t > 0)
            def _():
                wait_scatter(j0 - 1, buf1, s1)

            gather(j1, buf1, g1)
            wait_gather(j0, buf0, g0)
            scatter(j0, buf0, s0)

            @pl.when(t < n_pairs - 1)
            def _():
                wait_scatter(j0, buf0, s0)
                gather(j0 + 2, buf0, g0)

            wait_gather(j1, buf1, g1)
            scatter(j1, buf1, s1)
            return carry

        lax.fori_loop(0, n_pairs, pair_body, 0)
        wait_scatter(n_chunks - 2, buf0, s0)
        wait_scatter(n_chunks - 1, buf1, s1)

    return emb


def kernel(x, embedding_table):
    b, s = x.shape
    b_total = b * s
    idx = x.T.reshape(-1).astype(jnp.int32).reshape(NW, b_total // (NW * CHUNK), CHUNK)
    out = _make_emb_kernel(b_total)(idx, embedding_table)
    return out.reshape(s, b, D_MODEL).transpose(1, 0, 2)


# speedup vs baseline: 4.0700x; 1.0058x over previous
import functools

import jax
import jax.numpy as jnp
from jax import lax
from jax.experimental import pallas as pl
from jax.experimental.pallas import tpu as pltpu
from jax.experimental.pallas import tpu_sc as plsc

D_MODEL = 512
NC = 2
NS = 16
NW = NC * NS
CHUNK = 80


def _make_emb_kernel(b_total: int):
    b_per_w = b_total // NW
    n_chunks = b_per_w // CHUNK
    n_pairs = n_chunks // 2
    mesh = plsc.VectorSubcoreMesh(core_axis_name="c", subcore_axis_name="s")

    @functools.partial(
        pl.kernel,
        mesh=mesh,
        out_type=jax.ShapeDtypeStruct((b_total, D_MODEL), jnp.float32),
        scratch_types=[
            pltpu.VMEM((n_chunks, CHUNK), jnp.int32),
            pltpu.VMEM((CHUNK, D_MODEL), jnp.float32),
            pltpu.VMEM((CHUNK, D_MODEL), jnp.float32),
            pltpu.SemaphoreType.DMA,
            pltpu.SemaphoreType.DMA,
            pltpu.SemaphoreType.DMA,
            pltpu.SemaphoreType.DMA,
        ],
    )
    def emb(idx_hbm, table_hbm, out_hbm, idx_v, buf0, buf1, g0, g1, s0, s1):
        wid = lax.axis_index("s") * NC + lax.axis_index("c")
        pltpu.sync_copy(idx_hbm.at[wid], idx_v)
        base = wid * b_per_w

        def gather(j, buf, sem):
            pltpu.async_copy(table_hbm.at[idx_v.at[j]], buf, sem)

        def wait_gather(j, buf, sem):
            pltpu.make_async_copy(table_hbm.at[idx_v.at[j]], buf, sem).wait()

        def scatter(j, buf, sem):
            pltpu.async_copy(buf, out_hbm.at[pl.ds(base + j * CHUNK, CHUNK)], sem)

        def wait_scatter(j, buf, sem):
            pltpu.make_async_copy(
                buf, out_hbm.at[pl.ds(base + j * CHUNK, CHUNK)], sem
            ).wait()

        gather(0, buf0, g0)

        def pair_body(t, carry):
            j0 = 2 * t
            j1 = j0 + 1

            @pl.when(t > 0)
            def _():
                wait_scatter(j0 - 1, buf1, s1)

            gather(j1, buf1, g1)
            wait_gather(j0, buf0, g0)
            scatter(j0, buf0, s0)

            @pl.when(t < n_pairs - 1)
            def _():
                wait_scatter(j0, buf0, s0)
                gather(j0 + 2, buf0, g0)

            wait_gather(j1, buf1, g1)
            scatter(j1, buf1, s1)
            return carry

        lax.fori_loop(0, n_pairs, pair_body, 0)
        wait_scatter(n_chunks - 2, buf0, s0)
        wait_scatter(n_chunks - 1, buf1, s1)

    return emb


def kernel(x, embedding_table):
    b, s = x.shape
    b_total = b * s
    idx = x.T.reshape(-1).astype(jnp.int32).reshape(NW, b_total // (NW * CHUNK), CHUNK)
    out = _make_emb_kernel(b_total)(idx, embedding_table)
    return out.reshape(s, b, D_MODEL).transpose(1, 0, 2)
